# baseline (speedup 1.0000x reference)
import jax
import jax.numpy as jnp
from jax import lax
from jax.experimental import pallas as pl
from jax.experimental.pallas import tpu as pltpu

_TOKENS = 32768
_NUM_OPS = 64
_WIDTH = 64
_BLK = 512


def _tc_body(t_ref, mv_ref, mk_ref, out_ref):
    t = t_ref[...]
    mv = mv_ref[...]
    mk = mk_ref[...]
    ok = (t[:, None, :] == mv[None, :, :]) | (mk[None, :, :] != jnp.int32(0))
    viol = jnp.sum(
        jnp.where(ok, jnp.int32(0), jnp.int32(1)), axis=2, dtype=jnp.int32
    )
    iota_o = jax.lax.broadcasted_iota(jnp.int32, (_BLK, _NUM_OPS), 1)
    out_ref[...] = jnp.sum(
        jnp.where(viol == jnp.int32(0), iota_o, jnp.int32(0)),
        axis=1,
        dtype=jnp.int32,
    )


def _match_tc(t32, mv32, mk32, interpret=False):
    return pl.pallas_call(
        _tc_body,
        grid=(_TOKENS // _BLK,),
        in_specs=[
            pl.BlockSpec((_BLK, _WIDTH), lambda i: (i, jnp.int32(0))),
            pl.BlockSpec((_NUM_OPS, _WIDTH), lambda i: (jnp.int32(0), jnp.int32(0))),
            pl.BlockSpec((_NUM_OPS, _WIDTH), lambda i: (jnp.int32(0), jnp.int32(0))),
        ],
        out_specs=pl.BlockSpec((_BLK,), lambda i: (i,)),
        out_shape=jax.ShapeDtypeStruct((_TOKENS,), jnp.int32),
        interpret=interpret,
    )(t32, mv32, mk32)


def kernel(tensor, match_values, channel_masks, interpret=False):
    t32 = tensor.astype(jnp.int32)
    mv32 = match_values.astype(jnp.int32)
    mk32 = channel_masks.astype(jnp.int32)
    out = _match_tc(t32, mv32, mk32, interpret=interpret)
    return out.astype(tensor.dtype)


# speedup vs baseline: 6.6246x; 6.6246x over previous
import functools

import numpy as np
import jax
import jax.numpy as jnp
from jax import lax
from jax.experimental import pallas as pl
from jax.experimental.pallas import tpu as pltpu
from jax.experimental.pallas import tpu_sc as plsc

_TOKENS = 32768
_NUM_OPS = 64
_WIDTH = 64
_NC = 2
_NS = 16
_NW = _NC * _NS
_TPW = _TOKENS // _NW
_LANES = 16
_GROUPS = _TPW // _LANES

_POW2 = (1 << np.arange(32, dtype=np.uint64)).astype(np.uint32).view(np.int32)


def _lut_body(mv_ref, mk_ref, out_ref):
    mv = mv_ref[...]
    mk = mk_ref[...]
    iota_v = lax.broadcasted_iota(jnp.int32, (_NUM_OPS, _WIDTH, _NUM_OPS), 2)
    cond = (mv[:, :, None] == iota_v) | (mk[:, :, None] != jnp.int32(0))
    iota_o = lax.broadcasted_iota(jnp.int32, (32, 1, 1), 0)
    w = jnp.left_shift(jnp.int32(1), iota_o)
    lo = jnp.sum(jnp.where(cond[:32], w, jnp.int32(0)), axis=0, dtype=jnp.int32)
    hi = jnp.sum(jnp.where(cond[32:], w, jnp.int32(0)), axis=0, dtype=jnp.int32)
    out_ref[...] = jnp.stack([lo, hi], axis=0)


def _build_lut(mv32, mk32):
    return pl.pallas_call(
        _lut_body,
        out_shape=jax.ShapeDtypeStruct((2, _WIDTH, _NUM_OPS), jnp.int32),
    )(mv32, mk32)


def _bit_pos(acc):
    is31 = acc == jnp.int32(np.int32(-(2**31)))
    f = acc.astype(jnp.float32)
    bits = plsc.bitcast(f, jnp.int32)
    e = ((bits >> jnp.int32(23)) & jnp.int32(0xFF)) - jnp.int32(127)
    return jnp.where(is31, jnp.int32(31), e)


def _sc_body(t_hbm, lut_hbm, out_hbm, chunk, lut_lo, lut_hi, outv):
    wid = lax.axis_index("s") * np.int32(_NC) + lax.axis_index("c")
    base = wid * np.int32(_TPW)
    pltpu.sync_copy(lut_hbm.at[np.int32(0)], lut_lo)
    pltpu.sync_copy(lut_hbm.at[np.int32(1)], lut_hi)
    pltpu.sync_copy(t_hbm.at[pl.ds(base * np.int32(_WIDTH), _TPW * _WIDTH)], chunk)

    iota = lax.iota(jnp.int32, _LANES)
    iota_w = iota * np.int32(_WIDTH)

    @pl.loop(jnp.int32(0), jnp.int32(_GROUPS), step=jnp.int32(1))
    def _per_group(g):
        goff = g * np.int32(_LANES * _WIDTH)
        acc_lo = jnp.full((_LANES,), np.int32(-1), jnp.int32)
        acc_hi = jnp.full((_LANES,), np.int32(-1), jnp.int32)
        for c in range(_WIDTH):
            idx_t = iota_w + (goff + np.int32(c))
            v = plsc.load_gather(chunk, [idx_t])
            lidx = v + np.int32(c * _NUM_OPS)
            acc_lo = acc_lo & plsc.load_gather(lut_lo, [lidx])
            acc_hi = acc_hi & plsc.load_gather(lut_hi, [lidx])
        res = jnp.where(
            acc_lo != jnp.int32(0), _bit_pos(acc_lo), jnp.int32(32) + _bit_pos(acc_hi)
        )
        outv[pl.ds(g * np.int32(_LANES), _LANES)] = res

    pltpu.sync_copy(outv, out_hbm.at[pl.ds(base, _TPW)])


@functools.partial(
    pl.kernel,
    out_type=jax.ShapeDtypeStruct((_TOKENS,), jnp.int32),
    mesh=plsc.VectorSubcoreMesh(core_axis_name="c", subcore_axis_name="s"),
    scratch_types=[
        pltpu.VMEM((_TPW * _WIDTH,), jnp.int32),
        pltpu.VMEM((_WIDTH * _NUM_OPS,), jnp.int32),
        pltpu.VMEM((_WIDTH * _NUM_OPS,), jnp.int32),
        pltpu.VMEM((_TPW,), jnp.int32),
    ],
    compiler_params=pltpu.CompilerParams(needs_layout_passes=False),
)
def _sc_match(t_hbm, lut_hbm, out_hbm, chunk, lut_lo, lut_hi, outv):
    _sc_body(t_hbm, lut_hbm, out_hbm, chunk, lut_lo, lut_hi, outv)


def kernel(tensor, match_values, channel_masks):
    t32 = tensor.astype(jnp.int32)
    mv32 = match_values.astype(jnp.int32)
    mk32 = channel_masks.astype(jnp.int32)
    lut = _build_lut(mv32, mk32)
    out = _sc_match(t32.reshape(-1), lut.reshape(2, _WIDTH * _NUM_OPS))
    return out.astype(tensor.dtype)
